# baseline (speedup 1.0000x reference)
import functools

import jax
import jax.numpy as jnp
from jax import lax
from jax.experimental import pallas as pl
from jax.experimental.pallas import tpu as pltpu
from jax.experimental.pallas import tpu_sc as plsc

_TOK = 50
_EMB = 20
_HID = 60
_L = 16
_NGRP = 4


def _body(x_hbm, table_hbm, w1_hbm, b1_hbm, w2_hbm, b2_hbm, out_hbm,
          idx_v, e_v, w1_v, b1_v, w2_v, h_v, res_v, allp_v, b2_v,
          shared, sem):
    c = lax.axis_index("c")
    s = lax.axis_index("s")
    active = jnp.logical_and(c == 0, s < _NGRP)

    @pl.when(active)
    def _compute():
        pltpu.sync_copy(x_hbm, idx_v.at[pl.ds(0, _TOK)])
        iota = lax.iota(jnp.int32, _L)
        tok = s.astype(jnp.int32) * _L + iota
        tokc = jnp.minimum(tok, _TOK - 1)
        myidx = plsc.load_gather(idx_v, [tokc])
        gather = pltpu.async_copy(table_hbm.at[myidx], e_v, sem)
        pltpu.sync_copy(w1_hbm, w1_v)
        pltpu.sync_copy(b1_hbm, b1_v)
        pltpu.sync_copy(w2_hbm, w2_v)
        gather.wait()

        eT = [plsc.load_gather(e_v, [iota, jnp.full((_L,), k, jnp.int32)])
              for k in range(_EMB)]

        kc = [jnp.full((_L,), k, jnp.int32) for k in range(_EMB)]

        def p1(j, m):
            j_bc = jnp.zeros((_L,), jnp.int32) + j
            acc = plsc.load_gather(b1_v, [j_bc])
            for k in range(_EMB):
                acc = acc + plsc.load_gather(w1_v, [j_bc, kc[k]]) * eT[k]
            h_v[j] = acc
            return jnp.maximum(m, acc)

        m = lax.fori_loop(0, _HID, p1, jnp.full((_L,), -3e38, jnp.float32))

        w2base = tokc * _HID
        zero16 = jnp.zeros((_L,), jnp.int32)

        def p2(j, carry):
            den, num = carry
            p = jnp.exp(h_v[j] - m)
            wv = plsc.load_gather(w2_v, [zero16, w2base + j])
            return den + p, num + p * wv

        den, num = lax.fori_loop(
            0, _HID, p2,
            (jnp.zeros((_L,), jnp.float32), jnp.zeros((_L,), jnp.float32)))

        contrib = jnp.where(tok < _TOK, num / den, jnp.zeros((_L,), jnp.float32))
        res_v[...] = contrib
        pltpu.sync_copy(res_v, shared.at[s])

    plsc.subcore_barrier()

    @pl.when(jnp.logical_and(c == 0, s == 0))
    def _finalize():
        pltpu.sync_copy(b2_hbm, b2_v)
        pltpu.sync_copy(shared, allp_v)
        tot = allp_v[0] + allp_v[1] + allp_v[2] + allp_v[3]
        b2_bc = plsc.load_gather(b2_v, [jnp.zeros((_L,), jnp.int32)])
        total = jnp.sum(tot) + b2_bc
        res_v[...] = jnp.maximum(total, jnp.zeros((_L,), jnp.float32))
        pltpu.sync_copy(res_v.at[pl.ds(0, 1)], out_hbm)


_fwd = functools.partial(
    pl.kernel,
    out_type=jax.ShapeDtypeStruct((1,), jnp.float32),
    mesh=plsc.VectorSubcoreMesh(core_axis_name="c", subcore_axis_name="s"),
    compiler_params=pltpu.CompilerParams(
        needs_layout_passes=False, use_tc_tiling_on_sc=False),
    scratch_types=[
        pltpu.VMEM((64,), jnp.int32),
        pltpu.VMEM((_L, _EMB), jnp.float32),
        pltpu.VMEM((_HID, _EMB), jnp.float32),
        pltpu.VMEM((_HID,), jnp.float32),
        pltpu.VMEM((1, _TOK * _HID), jnp.float32),
        pltpu.VMEM((_HID, _L), jnp.float32),
        pltpu.VMEM((_L,), jnp.float32),
        pltpu.VMEM((_NGRP, _L), jnp.float32),
        pltpu.VMEM((1,), jnp.float32),
        pltpu.VMEM_SHARED((_NGRP, _L), jnp.float32),
        pltpu.SemaphoreType.DMA,
    ],
)(_body)


def kernel(x, table, W1, b1, W2, b2):
    return _fwd(x.astype(jnp.int32), table, W1, b1, W2, b2)


# speedup vs baseline: 2.1480x; 2.1480x over previous
import functools

import jax
import jax.numpy as jnp
from jax import lax
from jax.experimental import pallas as pl
from jax.experimental.pallas import tpu as pltpu
from jax.experimental.pallas import tpu_sc as plsc

_TOK = 50
_EMB = 20
_HID = 60
_L = 16
_NGRP = 4


def _body(x_hbm, table_hbm, w1_hbm, b1_hbm, w2_hbm, b2_hbm,
          out_hbm, scr_hbm,
          idx_v, e_v, w1_v, b1_v, w2_v, h_v, res_v, allp_v, b2_v, sem):
    c = lax.axis_index("c")
    s = lax.axis_index("s")
    active = jnp.logical_and(c == 0, s < _NGRP)

    @pl.when(active)
    def _compute():
        pltpu.sync_copy(x_hbm, idx_v.at[pl.ds(0, _TOK)])
        iota = lax.iota(jnp.int32, _L)
        tok = s.astype(jnp.int32) * _L + iota
        tokc = jnp.minimum(tok, _TOK - 1)
        myidx = plsc.load_gather(idx_v, [tokc])
        rem = jnp.bitwise_and(myidx, 7)
        handles = []
        for t in range(_L):
            base = pl.multiple_of(jnp.bitwise_and(myidx[t], ~7), 8)
            handles.append(pltpu.async_copy(
                table_hbm.at[pl.ds(base, 8)], e_v.at[t], sem))
        pltpu.sync_copy(w1_hbm, w1_v)
        pltpu.sync_copy(b1_hbm, b1_v)
        pltpu.sync_copy(w2_hbm, w2_v)
        for h in handles:
            h.wait()

        eT = [plsc.load_gather(e_v, [iota, rem, jnp.full((_L,), k, jnp.int32)])
              for k in range(_EMB)]

        kc = [jnp.full((_L,), k, jnp.int32) for k in range(_EMB)]

        def p1(j, m):
            j_bc = jnp.zeros((_L,), jnp.int32) + j
            acc = plsc.load_gather(b1_v, [j_bc])
            for k in range(_EMB):
                acc = acc + plsc.load_gather(w1_v, [j_bc, kc[k]]) * eT[k]
            h_v[j] = acc
            return jnp.maximum(m, acc)

        m = lax.fori_loop(0, _HID, p1, jnp.full((_L,), -3e38, jnp.float32))

        w2base = tokc * _HID
        zero16 = jnp.zeros((_L,), jnp.int32)

        def p2(j, carry):
            den, num = carry
            p = jnp.exp(h_v[j] - m)
            wv = plsc.load_gather(w2_v, [zero16, w2base + j])
            return den + p, num + p * wv

        den, num = lax.fori_loop(
            0, _HID, p2,
            (jnp.zeros((_L,), jnp.float32), jnp.zeros((_L,), jnp.float32)))

        contrib = jnp.where(tok < _TOK, num / den, jnp.zeros((_L,), jnp.float32))
        res_v[...] = contrib
        pltpu.sync_copy(res_v, scr_hbm.at[s])

    plsc.subcore_barrier()

    @pl.when(jnp.logical_and(c == 0, s == 0))
    def _finalize():
        pltpu.sync_copy(b2_hbm, b2_v)
        pltpu.sync_copy(scr_hbm, allp_v)
        tot = allp_v[0] + allp_v[1] + allp_v[2] + allp_v[3]
        b2_bc = plsc.load_gather(b2_v, [jnp.zeros((_L,), jnp.int32)])
        total = jnp.sum(tot) + b2_bc
        res_v[...] = jnp.maximum(total, jnp.zeros((_L,), jnp.float32))
        pltpu.sync_copy(res_v.at[pl.ds(0, 1)], out_hbm)


_fwd = functools.partial(
    pl.kernel,
    out_type=(jax.ShapeDtypeStruct((1,), jnp.float32),
              jax.ShapeDtypeStruct((_NGRP, _L), jnp.float32)),
    mesh=plsc.VectorSubcoreMesh(core_axis_name="c", subcore_axis_name="s"),
    compiler_params=pltpu.CompilerParams(needs_layout_passes=False),
    scratch_types=[
        pltpu.VMEM((64,), jnp.int32),
        pltpu.VMEM((_L, 8, _EMB), jnp.float32),
        pltpu.VMEM((_HID, _EMB), jnp.float32),
        pltpu.VMEM((_HID,), jnp.float32),
        pltpu.VMEM((1, _TOK * _HID), jnp.float32),
        pltpu.VMEM((_HID, _L), jnp.float32),
        pltpu.VMEM((_L,), jnp.float32),
        pltpu.VMEM((_NGRP, _L), jnp.float32),
        pltpu.VMEM((1,), jnp.float32),
        pltpu.SemaphoreType.DMA,
    ],
)(_body)


def kernel(x, table, W1, b1, W2, b2):
    out, _ = _fwd(x.astype(jnp.int32), table, W1, b1, W2, b2)
    return out


# speedup vs baseline: 2.2101x; 1.0289x over previous
import functools

import jax
import jax.numpy as jnp
from jax import lax
from jax.experimental import pallas as pl
from jax.experimental.pallas import tpu as pltpu
from jax.experimental.pallas import tpu_sc as plsc

_TOK = 50
_EMB = 20
_HID = 60
_L = 16
_NGRP = 4


def _body(x_hbm, table_hbm, w1_hbm, b1_hbm, w2_hbm, b2_hbm,
          out_hbm, scr_hbm,
          idx_v, e_v, w1_v, b1_v, w2_v, h_v, res_v, allp_v, b2_v, sem):
    c = lax.axis_index("c")
    s = lax.axis_index("s")
    active = jnp.logical_and(c == 0, s < _NGRP)

    @pl.when(active)
    def _compute():
        pltpu.sync_copy(x_hbm, idx_v.at[pl.ds(0, _TOK)])
        iota = lax.iota(jnp.int32, _L)
        tok = s.astype(jnp.int32) * _L + iota
        tokc = jnp.minimum(tok, _TOK - 1)
        myidx = plsc.load_gather(idx_v, [tokc])
        rem = jnp.bitwise_and(myidx, 7)
        handles = []
        for t in range(_L):
            base = pl.multiple_of(jnp.bitwise_and(myidx[t], ~7), 8)
            handles.append(pltpu.async_copy(
                table_hbm.at[pl.ds(base, 8)], e_v.at[t], sem))
        pltpu.sync_copy(w1_hbm, w1_v)
        pltpu.sync_copy(b1_hbm, b1_v)
        pltpu.sync_copy(w2_hbm, w2_v)
        for h in handles:
            h.wait()

        eT = [plsc.load_gather(e_v, [iota, rem, jnp.full((_L,), k, jnp.int32)])
              for k in range(_EMB)]

        kc = [jnp.full((_L,), k, jnp.int32) for k in range(_EMB)]

        def p1(j, m):
            j_bc = jnp.zeros((_L,), jnp.int32) + j
            acc = plsc.load_gather(b1_v, [j_bc])
            for k in range(_EMB):
                acc = acc + plsc.load_gather(w1_v, [j_bc, kc[k]]) * eT[k]
            h_v[j] = acc
            return jnp.maximum(m, acc)

        m = lax.fori_loop(0, _HID, p1, jnp.full((_L,), -3e38, jnp.float32))

        w2base = tokc * _HID
        zero16 = jnp.zeros((_L,), jnp.int32)

        def p2(j, carry):
            den, num = carry
            p = jnp.exp(h_v[j] - m)
            wv = plsc.load_gather(w2_v, [zero16, w2base + j])
            return den + p, num + p * wv

        den, num = lax.fori_loop(
            0, _HID, p2,
            (jnp.zeros((_L,), jnp.float32), jnp.zeros((_L,), jnp.float32)))

        contrib = jnp.where(tok < _TOK, num / den, jnp.zeros((_L,), jnp.float32))
        res_v[...] = contrib
        pltpu.sync_copy(res_v, scr_hbm.at[s])

    plsc.subcore_barrier()

    @pl.when(jnp.logical_and(c == 0, s == 0))
    def _finalize():
        pltpu.sync_copy(b2_hbm, b2_v)
        pltpu.sync_copy(scr_hbm, allp_v)
        tot = allp_v[0] + allp_v[1] + allp_v[2] + allp_v[3]
        b2_bc = plsc.load_gather(b2_v, [jnp.zeros((_L,), jnp.int32)])
        total = jnp.sum(tot) + b2_bc
        res_v[...] = jnp.maximum(total, jnp.zeros((_L,), jnp.float32))
        pltpu.sync_copy(res_v.at[pl.ds(0, 1)], out_hbm)


_fwd = functools.partial(
    pl.kernel,
    out_type=(jax.ShapeDtypeStruct((1,), jnp.float32),
              jax.ShapeDtypeStruct((_NGRP, _L), jnp.float32)),
    mesh=plsc.VectorSubcoreMesh(core_axis_name="c", subcore_axis_name="s",
                                num_cores=1),
    compiler_params=pltpu.CompilerParams(needs_layout_passes=False),
    scratch_types=[
        pltpu.VMEM((64,), jnp.int32),
        pltpu.VMEM((_L, 8, _EMB), jnp.float32),
        pltpu.VMEM((_HID, _EMB), jnp.float32),
        pltpu.VMEM((_HID,), jnp.float32),
        pltpu.VMEM((1, _TOK * _HID), jnp.float32),
        pltpu.VMEM((_HID, _L), jnp.float32),
        pltpu.VMEM((_L,), jnp.float32),
        pltpu.VMEM((_NGRP, _L), jnp.float32),
        pltpu.VMEM((1,), jnp.float32),
        pltpu.SemaphoreType.DMA,
    ],
)(_body)


def kernel(x, table, W1, b1, W2, b2):
    out, _ = _fwd(x.astype(jnp.int32), table, W1, b1, W2, b2)
    return out


# speedup vs baseline: 2.5200x; 1.1402x over previous
import functools

import jax
import jax.numpy as jnp
from jax import lax
from jax.experimental import pallas as pl
from jax.experimental.pallas import tpu as pltpu
from jax.experimental.pallas import tpu_sc as plsc

_TOK = 50
_EMB = 20
_HID = 60
_L = 16
_NGRP = 4


def _body(x_hbm, table_hbm, w1_hbm, b1_hbm, w2_hbm, b2_hbm,
          out_hbm, scr_hbm,
          idx_v, e_v, w1_v, b1_v, w2_v, h_v, res_v, allp_v, b2_v, sem):
    c = lax.axis_index("c")
    s = lax.axis_index("s")
    active = jnp.logical_and(c == 0, s < _NGRP)

    @pl.when(active)
    def _compute():
        pltpu.sync_copy(x_hbm, idx_v.at[pl.ds(0, _TOK)])
        iota = lax.iota(jnp.int32, _L)
        tok = s.astype(jnp.int32) * _L + iota
        tokc = jnp.minimum(tok, _TOK - 1)
        myidx = plsc.load_gather(idx_v, [tokc])
        rem = jnp.bitwise_and(myidx, 7)
        handles = []
        for t in range(_L):
            tid = jnp.right_shift(myidx[t], 3)
            handles.append(pltpu.async_copy(
                table_hbm.at[tid], e_v.at[t], sem))
        pltpu.sync_copy(w1_hbm, w1_v)
        pltpu.sync_copy(b1_hbm, b1_v)
        pltpu.sync_copy(w2_hbm, w2_v)
        for h in handles:
            h.wait()

        eT = [plsc.load_gather(e_v, [iota, rem, jnp.full((_L,), k, jnp.int32)])
              for k in range(_EMB)]

        kc = [jnp.full((_L,), k, jnp.int32) for k in range(_EMB)]

        def p1(j, m):
            j_bc = jnp.zeros((_L,), jnp.int32) + j
            acc = plsc.load_gather(b1_v, [j_bc])
            for k in range(_EMB):
                acc = acc + plsc.load_gather(w1_v, [j_bc, kc[k]]) * eT[k]
            h_v[j] = acc
            return jnp.maximum(m, acc)

        m = lax.fori_loop(0, _HID, p1, jnp.full((_L,), -3e38, jnp.float32))

        w2base = tokc * _HID
        zero16 = jnp.zeros((_L,), jnp.int32)

        def p2(j, carry):
            den, num = carry
            p = jnp.exp(h_v[j] - m)
            wv = plsc.load_gather(w2_v, [zero16, w2base + j])
            return den + p, num + p * wv

        den, num = lax.fori_loop(
            0, _HID, p2,
            (jnp.zeros((_L,), jnp.float32), jnp.zeros((_L,), jnp.float32)))

        contrib = jnp.where(tok < _TOK, num / den, jnp.zeros((_L,), jnp.float32))
        res_v[...] = contrib
        pltpu.sync_copy(res_v, scr_hbm.at[s])

    plsc.subcore_barrier()

    @pl.when(jnp.logical_and(c == 0, s == 0))
    def _finalize():
        pltpu.sync_copy(b2_hbm, b2_v)
        pltpu.sync_copy(scr_hbm, allp_v)
        tot = allp_v[0] + allp_v[1] + allp_v[2] + allp_v[3]
        b2_bc = plsc.load_gather(b2_v, [jnp.zeros((_L,), jnp.int32)])
        total = jnp.sum(tot) + b2_bc
        res_v[...] = jnp.maximum(total, jnp.zeros((_L,), jnp.float32))
        pltpu.sync_copy(res_v.at[pl.ds(0, 1)], out_hbm)


_fwd = functools.partial(
    pl.kernel,
    out_type=(jax.ShapeDtypeStruct((1,), jnp.float32),
              jax.ShapeDtypeStruct((_NGRP, _L), jnp.float32)),
    mesh=plsc.VectorSubcoreMesh(core_axis_name="c", subcore_axis_name="s",
                                num_cores=1),
    compiler_params=pltpu.CompilerParams(needs_layout_passes=False),
    scratch_types=[
        pltpu.VMEM((64,), jnp.int32),
        pltpu.VMEM((_L, 8, _EMB), jnp.float32),
        pltpu.VMEM((_HID, _EMB), jnp.float32),
        pltpu.VMEM((_HID,), jnp.float32),
        pltpu.VMEM((1, _TOK * _HID), jnp.float32),
        pltpu.VMEM((_HID, _L), jnp.float32),
        pltpu.VMEM((_L,), jnp.float32),
        pltpu.VMEM((_NGRP, _L), jnp.float32),
        pltpu.VMEM((1,), jnp.float32),
        pltpu.SemaphoreType.DMA,
    ],
)(_body)


def kernel(x, table, W1, b1, W2, b2):
    table3 = table.reshape(table.shape[0] // 8, 8, table.shape[1])
    out, _ = _fwd(x.astype(jnp.int32), table3, W1, b1, W2, b2)
    return out
